# baseline (speedup 1.0000x reference)
import jax
import jax.numpy as jnp
import numpy as np
from jax import lax
from jax.experimental import pallas as pl
from jax.experimental.pallas import tpu as pltpu
from jax.experimental.pallas import tpu_sc as plsc

B = 4096
F = 26
D = 32
FD = F * D
P = F * (F - 1) // 2

NC = 2
NS = 16
NW = NC * NS
RW = B // NW
NG = RW // 16

_ti, _tj = np.triu_indices(F, k=1)
_PAIR_ID = {(int(a), int(b)): k for k, (a, b) in enumerate(zip(_ti, _tj))}

_FB = [(s, min(s + 7, F)) for s in range(0, F, 7)]
MAXR = 7 * D

_ORDER = []
for _ai in range(len(_FB)):
    _bs = list(range(_ai + 1, len(_FB)))
    if _ai % 2 == 1:
        _bs = _bs[::-1]
    _ORDER.append((_ai, _ai))
    _ORDER.extend((_ai, _b) for _b in _bs)


def _body(x_hbm, out_hbm, a_slab, b_slab, out_vmem):
    wid = lax.axis_index("s") * NC + lax.axis_index("c")

    def do_block_pair(ai, bi):
        alo, ahi = _FB[ai]
        blo, bhi = _FB[bi]
        pairs = [(i, j) for i in range(alo, ahi)
                 for j in range(blo, bhi) if i < j]
        slab_b = a_slab if bi == ai else b_slab

        def group_body(g, carry2):
            lane0 = g * 16

            def d_body(d, accs, _pairs=pairs):
                va = {f: a_slab[(f - alo) * D + d, pl.ds(lane0, 16)]
                      for f in range(alo, ahi)}
                vb = va if bi == ai else {
                    f: slab_b[(f - blo) * D + d, pl.ds(lane0, 16)]
                    for f in range(blo, bhi)}
                return tuple(a + va[i] * vb[j]
                             for a, (i, j) in zip(accs, _pairs))

            init = tuple(jnp.zeros((16,), jnp.float32) for _ in pairs)
            accs = plsc.parallel_loop(0, D, carry=init, unroll=4)(d_body)
            for acc, (i, j) in zip(accs, pairs):
                out_vmem[_PAIR_ID[(i, j)], pl.ds(lane0, 16)] = acc
            return carry2

        lax.fori_loop(0, NG, group_body, 0)

    loaded_a = loaded_b = None
    for ai, bi in _ORDER:
        alo, ahi = _FB[ai]
        blo, bhi = _FB[bi]
        if loaded_a != ai:
            pltpu.sync_copy(x_hbm.at[wid, pl.ds(alo * D, (ahi - alo) * D), :],
                            a_slab.at[pl.ds(0, (ahi - alo) * D)])
            loaded_a = ai
        if bi != ai and loaded_b != bi:
            pltpu.sync_copy(x_hbm.at[wid, pl.ds(blo * D, (bhi - blo) * D), :],
                            b_slab.at[pl.ds(0, (bhi - blo) * D)])
            loaded_b = bi
        do_block_pair(ai, bi)

    pltpu.sync_copy(out_vmem, out_hbm.at[wid])


_mesh = plsc.VectorSubcoreMesh(core_axis_name="c", subcore_axis_name="s",
                               num_cores=NC, num_subcores=NS)

_sc_call = pl.kernel(
    _body,
    out_type=jax.ShapeDtypeStruct((NW, P, RW), jnp.float32),
    mesh=_mesh,
    scratch_types=[
        pltpu.VMEM((MAXR, RW), jnp.float32),
        pltpu.VMEM((MAXR, RW), jnp.float32),
        pltpu.VMEM((P, RW), jnp.float32),
    ],
)


@jax.jit
def kernel(inputs):
    x_r = inputs.reshape(NW, RW, FD).transpose(0, 2, 1)
    out_r = _sc_call(x_r)
    return out_r.transpose(0, 2, 1).reshape(B, P)


# speedup vs baseline: 1.0511x; 1.0511x over previous
import jax
import jax.numpy as jnp
import numpy as np
from jax import lax
from jax.experimental import pallas as pl
from jax.experimental.pallas import tpu as pltpu
from jax.experimental.pallas import tpu_sc as plsc

B = 4096
F = 26
D = 32
FD = F * D
P = F * (F - 1) // 2

NC = 2
NS = 16
NW = NC * NS
RW = B // NW
NG = RW // 16

_ti, _tj = np.triu_indices(F, k=1)
_PAIR_ID = {(int(a), int(b)): k for k, (a, b) in enumerate(zip(_ti, _tj))}

_FB = [(s, min(s + 6, F)) for s in range(0, F, 6)]
MAXR = 6 * D

_ORDER = []
for _ai in range(len(_FB)):
    _bs = list(range(_ai + 1, len(_FB)))
    if _ai % 2 == 1:
        _bs = _bs[::-1]
    _ORDER.append((_ai, _ai))
    _ORDER.extend((_ai, _b) for _b in _bs)


def _body(x_hbm, out_hbm, a_slab, b_slab, out_vmem):
    wid = lax.axis_index("s") * NC + lax.axis_index("c")

    def do_block_pair(ai, bi):
        alo, ahi = _FB[ai]
        blo, bhi = _FB[bi]
        pairs = [(i, j) for i in range(alo, ahi)
                 for j in range(blo, bhi) if i < j]
        slab_b = a_slab if bi == ai else b_slab

        def group_body(g, carry2):
            lane0 = g * 16

            def d_body(d, accs, _pairs=pairs):
                va = {f: a_slab[(f - alo) * D + d, pl.ds(lane0, 16)]
                      for f in range(alo, ahi)}
                vb = va if bi == ai else {
                    f: slab_b[(f - blo) * D + d, pl.ds(lane0, 16)]
                    for f in range(blo, bhi)}
                return tuple(a + va[i] * vb[j]
                             for a, (i, j) in zip(accs, _pairs))

            init = tuple(jnp.zeros((16,), jnp.float32) for _ in pairs)
            accs = plsc.parallel_loop(0, D, carry=init, unroll=4)(d_body)
            for acc, (i, j) in zip(accs, pairs):
                out_vmem[_PAIR_ID[(i, j)], pl.ds(lane0, 16)] = acc
            return carry2

        lax.fori_loop(0, NG, group_body, 0)

    loaded_a = loaded_b = None
    for ai, bi in _ORDER:
        alo, ahi = _FB[ai]
        blo, bhi = _FB[bi]
        if loaded_a != ai:
            pltpu.sync_copy(x_hbm.at[wid, pl.ds(alo * D, (ahi - alo) * D), :],
                            a_slab.at[pl.ds(0, (ahi - alo) * D)])
            loaded_a = ai
        if bi != ai and loaded_b != bi:
            pltpu.sync_copy(x_hbm.at[wid, pl.ds(blo * D, (bhi - blo) * D), :],
                            b_slab.at[pl.ds(0, (bhi - blo) * D)])
            loaded_b = bi
        do_block_pair(ai, bi)

    pltpu.sync_copy(out_vmem, out_hbm.at[wid])


_mesh = plsc.VectorSubcoreMesh(core_axis_name="c", subcore_axis_name="s",
                               num_cores=NC, num_subcores=NS)

_sc_call = pl.kernel(
    _body,
    out_type=jax.ShapeDtypeStruct((NW, P, RW), jnp.float32),
    mesh=_mesh,
    scratch_types=[
        pltpu.VMEM((MAXR, RW), jnp.float32),
        pltpu.VMEM((MAXR, RW), jnp.float32),
        pltpu.VMEM((P, RW), jnp.float32),
    ],
)


@jax.jit
def kernel(inputs):
    x_r = inputs.reshape(NW, RW, FD).transpose(0, 2, 1)
    out_r = _sc_call(x_r)
    return out_r.transpose(0, 2, 1).reshape(B, P)
